# baseline (speedup 1.0000x reference)
import functools

import jax
import jax.numpy as jnp
from jax import lax
from jax.experimental import pallas as pl
from jax.experimental.pallas import tpu as pltpu
from jax.experimental.pallas import tpu_sc as plsc

N_FIELDS = 26
VOCAB = 100000
EMBED_DIM = 16
BATCH = 16384

NUM_CORES = 2
NUM_SUBCORES = 16
NUM_WORKERS = NUM_CORES * NUM_SUBCORES
PAIRS = N_FIELDS * EMBED_DIM
PAIRS_PER_WORKER = PAIRS // NUM_WORKERS
IDX_CHUNK = 4096
LANES = 16


def _body(xt_hbm, tab_hbm, out_hbm, row_v, idx_v, ov0_v, ov1_v,
          semr, semi, semw0, semw1):
    wid = lax.axis_index("s") * NUM_CORES + lax.axis_index("c")
    p0 = wid * PAIRS_PER_WORKER
    ov_bufs = (ov0_v, ov1_v)
    ov_sems = (semw0, semw1)
    QROWS = BATCH // 128 // 4

    def out_slice(p, q):
        return out_hbm.at[p >> 4, (p & 15) >> 3, pl.ds(q * QROWS, QROWS),
                          p & 7, :]

    def pair_step(k, _):
        p = p0 + k
        f = p >> 4
        cr = pltpu.async_copy(tab_hbm.at[f, p & 15], row_v, semr)

        @pl.when((k == 0) | ((p & 15) == 0))
        def _stage_idx():
            pltpu.async_copy(xt_hbm.at[f], idx_v, semi).wait()

        cr.wait()
        for q in range(4):
            buf = ov_bufs[q & 1]
            if q < 2:
                @pl.when(k > 0)
                def _drain():
                    pltpu.make_async_copy(buf, out_slice(p, q + 2),
                                          ov_sems[q & 1]).wait()
            else:
                pltpu.make_async_copy(buf, out_slice(p, q - 2),
                                      ov_sems[q & 1]).wait()

            def gat_step(i, _):
                for j in range(8):
                    sl = pl.ds(((q * QROWS + i) * 8 + j) * LANES, LANES)
                    val = plsc.load_gather(row_v, [idx_v[sl]])
                    buf[i, pl.ds(j * LANES, LANES)] = val
                return ()

            lax.fori_loop(0, QROWS, gat_step, ())
            pltpu.async_copy(buf, out_slice(p, q), ov_sems[q & 1])
        return ()

    lax.fori_loop(0, PAIRS_PER_WORKER, pair_step, ())
    for q in (2, 3):
        pltpu.make_async_copy(ov_bufs[q & 1],
                              out_slice(p0 + PAIRS_PER_WORKER - 1, q),
                              ov_sems[q & 1]).wait()


@functools.partial(jax.jit, static_argnames=())
def kernel(x, tables):
    xt = x.astype(jnp.int32).T
    tab_t = tables.transpose(0, 2, 1)

    mesh = plsc.VectorSubcoreMesh(core_axis_name="c", subcore_axis_name="s")
    out5 = pl.kernel(
        _body,
        out_type=jax.ShapeDtypeStruct(
            (N_FIELDS, 2, BATCH // 128, 8, 128), jnp.float32),
        mesh=mesh,
        scratch_types=[
            pltpu.VMEM((VOCAB,), jnp.float32),
            pltpu.VMEM((BATCH,), jnp.int32),
            pltpu.VMEM((BATCH // 128 // 4, 128), jnp.float32),
            pltpu.VMEM((BATCH // 128 // 4, 128), jnp.float32),
            pltpu.SemaphoreType.DMA,
            pltpu.SemaphoreType.DMA,
            pltpu.SemaphoreType.DMA,
            pltpu.SemaphoreType.DMA,
        ],
        compiler_params=pltpu.CompilerParams(use_tc_tiling_on_sc=True,
                                             needs_layout_passes=False),
    )(xt, tab_t)
    return out5.transpose(2, 4, 0, 1, 3).reshape(BATCH, N_FIELDS, EMBED_DIM)


# speedup vs baseline: 1.0014x; 1.0014x over previous
import functools

import jax
import jax.numpy as jnp
from jax import lax
from jax.experimental import pallas as pl
from jax.experimental.pallas import tpu as pltpu
from jax.experimental.pallas import tpu_sc as plsc

N_FIELDS = 26
VOCAB = 100000
EMBED_DIM = 16
BATCH = 16384

NUM_CORES = 2
NUM_SUBCORES = 16
NUM_WORKERS = NUM_CORES * NUM_SUBCORES
PAIRS = N_FIELDS * EMBED_DIM
PAIRS_PER_WORKER = PAIRS // NUM_WORKERS
LANES = 16


def _body(xt_hbm, tab_hbm, out_hbm, row_v, idx_v, ov0_v, ov1_v,
          semr, semi, semw0, semw1):
    wid = lax.axis_index("s") * NUM_CORES + lax.axis_index("c")
    p0 = wid * PAIRS_PER_WORKER
    ov_bufs = (ov0_v, ov1_v)
    ov_sems = (semw0, semw1)
    QROWS = BATCH // 128 // 4

    def out_slice(p, q):
        return out_hbm.at[p >> 4, (p & 15) >> 3, pl.ds(q * QROWS, QROWS),
                          p & 7, :]

    def pair_step(k, _):
        p = p0 + k
        f = p >> 4
        cr = pltpu.async_copy(tab_hbm.at[f, p & 15], row_v, semr)

        @pl.when((k == 0) | ((p & 15) == 0))
        def _stage_idx():
            pltpu.async_copy(xt_hbm.at[f], idx_v, semi).wait()

        cr.wait()
        for q in range(4):
            buf = ov_bufs[q & 1]
            if q < 2:
                @pl.when(k > 0)
                def _drain():
                    pltpu.make_async_copy(buf, out_slice(p, q + 2),
                                          ov_sems[q & 1]).wait()
            else:
                pltpu.make_async_copy(buf, out_slice(p, q - 2),
                                      ov_sems[q & 1]).wait()

            def gat_step(i, _):
                for j in range(8):
                    sl = pl.ds(((q * QROWS + i) * 8 + j) * LANES, LANES)
                    val = plsc.load_gather(row_v, [idx_v[sl]])
                    buf[i, pl.ds(j * LANES, LANES)] = val
                return ()

            lax.fori_loop(0, QROWS, gat_step, ())
            pltpu.async_copy(buf, out_slice(p, q), ov_sems[q & 1])
        return ()

    lax.fori_loop(0, PAIRS_PER_WORKER, pair_step, ())
    for q in (2, 3):
        pltpu.make_async_copy(ov_bufs[q & 1],
                              out_slice(p0 + PAIRS_PER_WORKER - 1, q),
                              ov_sems[q & 1]).wait()


@functools.partial(jax.jit, static_argnames=())
def kernel(x, tables):
    xt = x.astype(jnp.int32).T
    tab_t = tables.transpose(0, 2, 1)

    mesh = plsc.VectorSubcoreMesh(core_axis_name="c", subcore_axis_name="s")
    out5 = pl.kernel(
        _body,
        out_type=jax.ShapeDtypeStruct(
            (N_FIELDS, 2, BATCH // 128, 8, 128), jnp.float32),
        mesh=mesh,
        scratch_types=[
            pltpu.VMEM((VOCAB,), jnp.float32),
            pltpu.VMEM((BATCH,), jnp.int32),
            pltpu.VMEM((BATCH // 128 // 4, 128), jnp.float32),
            pltpu.VMEM((BATCH // 128 // 4, 128), jnp.float32),
            pltpu.SemaphoreType.DMA,
            pltpu.SemaphoreType.DMA,
            pltpu.SemaphoreType.DMA,
            pltpu.SemaphoreType.DMA,
        ],
        compiler_params=pltpu.CompilerParams(use_tc_tiling_on_sc=True,
                                             needs_layout_passes=False),
    )(xt, tab_t)
    return out5.transpose(2, 4, 0, 1, 3).reshape(BATCH, N_FIELDS, EMBED_DIM)
